# baseline (speedup 1.0000x reference)
import jax
import jax.numpy as jnp
from jax.experimental import pallas as pl
from jax.experimental.pallas import tpu as pltpu

_B, _C, _H, _W = 4, 96, 384, 384
_HT = 32


def _pool_body(x_ref, m_ref, sums_ref, cnt_ref):
    b = pl.program_id(0)
    h = pl.program_id(1)

    @pl.when((b == 0) & (h == 0))
    def _init():
        sums_ref[...] = jnp.zeros_like(sums_ref)
        cnt_ref[0, 0] = jnp.float32(0.0)

    xb = x_ref[0]
    mb = m_ref[0].astype(jnp.float32)
    s1 = jnp.sum(xb * mb[None, :, :], axis=(1, 2))
    s0 = jnp.sum(xb, axis=(1, 2))
    sums_ref[...] += jnp.stack([s1, s0])
    cnt_ref[0, 0] += jnp.sum(mb)


def kernel(x, mask):
    B, C, H, W = x.shape
    grid = (B, H // _HT)
    sums, cnt = pl.pallas_call(
        _pool_body,
        grid=grid,
        in_specs=[
            pl.BlockSpec((1, C, _HT, W), lambda b, h: (b, 0, h, 0)),
            pl.BlockSpec((1, _HT, W), lambda b, h: (b, h, 0)),
        ],
        out_specs=[
            pl.BlockSpec((2, C), lambda b, h: (0, 0)),
            pl.BlockSpec(memory_space=pltpu.SMEM),
        ],
        out_shape=[
            jax.ShapeDtypeStruct((2, C), jnp.float32),
            jax.ShapeDtypeStruct((1, 1), jnp.float32),
        ],
    )(x, mask)
    n_ch = cnt[0, 0]
    n_tot = jnp.float32(B * H * W)
    ch = sums[0] / n_ch
    unch = (sums[1] - sums[0]) / (n_tot - n_ch)
    return (unch, ch)


# speedup vs baseline: 1.1173x; 1.1173x over previous
import jax
import jax.numpy as jnp
from jax.experimental import pallas as pl
from jax.experimental.pallas import tpu as pltpu

_HT = 64


def _pool_body(x_ref, m_ref, sums_ref, cnt_ref):
    h = pl.program_id(1)

    @pl.when(h == 0)
    def _init():
        sums_ref[...] = jnp.zeros_like(sums_ref)
        cnt_ref[0, 0, 0] = jnp.float32(0.0)

    xb = x_ref[0]
    mb = m_ref[0].astype(jnp.float32)
    s1 = jnp.sum(xb * mb[None, :, :], axis=(1, 2))
    s0 = jnp.sum(xb, axis=(1, 2))
    sums_ref[0] += jnp.stack([s1, s0])
    cnt_ref[0, 0, 0] += jnp.sum(mb)


def kernel(x, mask):
    B, C, H, W = x.shape
    grid = (B, H // _HT)
    sums, cnt = pl.pallas_call(
        _pool_body,
        grid=grid,
        in_specs=[
            pl.BlockSpec((1, C, _HT, W), lambda b, h: (b, 0, h, 0)),
            pl.BlockSpec((1, _HT, W), lambda b, h: (b, h, 0)),
        ],
        out_specs=[
            pl.BlockSpec((1, 2, C), lambda b, h: (b, 0, 0)),
            pl.BlockSpec((1, 1, 1), lambda b, h: (b, 0, 0), memory_space=pltpu.SMEM),
        ],
        out_shape=[
            jax.ShapeDtypeStruct((B, 2, C), jnp.float32),
            jax.ShapeDtypeStruct((B, 1, 1), jnp.float32),
        ],
        compiler_params=pltpu.CompilerParams(
            dimension_semantics=("parallel", "arbitrary"),
        ),
    )(x, mask)
    tot = jnp.sum(sums, axis=0)
    n_ch = jnp.sum(cnt)
    n_tot = jnp.float32(B * H * W)
    ch = tot[0] / n_ch
    unch = (tot[1] - tot[0]) / (n_tot - n_ch)
    return (unch, ch)


# speedup vs baseline: 1.1412x; 1.0214x over previous
import jax
import jax.numpy as jnp
from jax.experimental import pallas as pl
from jax.experimental.pallas import tpu as pltpu

_HT = 64


def _pool_body(x_ref, m_ref, sums_ref, cnt_ref):
    b = pl.program_id(0)
    h = pl.program_id(1)

    @pl.when((b == 0) & (h == 0))
    def _init():
        sums_ref[...] = jnp.zeros_like(sums_ref)
        cnt_ref[0, 0] = jnp.float32(0.0)

    xb = x_ref[0]
    mb = m_ref[0].astype(jnp.float32)
    s1 = jnp.sum(xb * mb[None, :, :], axis=(1, 2))
    s0 = jnp.sum(xb, axis=(1, 2))
    sums_ref[...] += jnp.stack([s1, s0])
    cnt_ref[0, 0] += jnp.sum(mb)


def kernel(x, mask):
    B, C, H, W = x.shape
    grid = (B, H // _HT)
    sums, cnt = pl.pallas_call(
        _pool_body,
        grid=grid,
        in_specs=[
            pl.BlockSpec((1, C, _HT, W), lambda b, h: (b, 0, h, 0)),
            pl.BlockSpec((1, _HT, W), lambda b, h: (b, h, 0)),
        ],
        out_specs=[
            pl.BlockSpec((2, C), lambda b, h: (0, 0)),
            pl.BlockSpec(memory_space=pltpu.SMEM),
        ],
        out_shape=[
            jax.ShapeDtypeStruct((2, C), jnp.float32),
            jax.ShapeDtypeStruct((1, 1), jnp.float32),
        ],
    )(x, mask)
    n_ch = cnt[0, 0]
    n_tot = jnp.float32(B * H * W)
    ch = sums[0] / n_ch
    unch = (sums[1] - sums[0]) / (n_tot - n_ch)
    return (unch, ch)
